# baseline (speedup 1.0000x reference)
import jax
import jax.numpy as jnp
from jax import lax
from jax.experimental import pallas as pl
from jax.experimental.pallas import tpu as pltpu
from jax.experimental.pallas import tpu_sc as plsc

B = 16384
EMBED_K = 64
FEAT = 128

NC = 2
NS = 16
NW = NC * NS
B_PER_W = B // NW
CHUNK = 128
N_CHUNKS = B_PER_W // CHUNK


def _sc_gather_f_body(items_hbm, f_hbm, gut_hbm, git_hbm,
                      effe_out, gamut_out, gamit_out, iidx_v, irows_v, isem):
    wid = lax.axis_index("s") * NC + lax.axis_index("c")
    base = wid * B_PER_W
    s_cols = pl.ds(base, B_PER_W)
    pltpu.sync_copy(items_hbm.at[s_cols], iidx_v)
    copies = []
    for c in range(N_CHUNKS):
        s = pl.ds(c * CHUNK, CHUNK)
        copies.append(pltpu.async_copy(f_hbm.at[iidx_v.at[s]], irows_v.at[s], isem))
    pltpu.sync_copy(gut_hbm.at[:, s_cols], gamut_out.at[:, s_cols])
    pltpu.sync_copy(git_hbm.at[:, s_cols], gamit_out.at[:, s_cols])
    for cp in copies:
        cp.wait()
    pltpu.sync_copy(irows_v, effe_out.at[s_cols])


def _sc_gather_f(items, f, gut, git):
    mesh = plsc.VectorSubcoreMesh(core_axis_name="c", subcore_axis_name="s")
    return pl.kernel(
        _sc_gather_f_body,
        out_type=(
            jax.ShapeDtypeStruct((B, FEAT), jnp.float32),
            jax.ShapeDtypeStruct((EMBED_K, B), jnp.float32),
            jax.ShapeDtypeStruct((EMBED_K, B), jnp.float32),
        ),
        mesh=mesh,
        scratch_types=[
            pltpu.VMEM((B_PER_W,), jnp.int32),
            pltpu.VMEM((B_PER_W, FEAT), jnp.float32),
            pltpu.SemaphoreType.DMA,
        ],
    )(items, f, gut, git)


def _sc_gather_tu_body(users_hbm, tu_hbm, theta_out, uidx_v, urows_v, usem):
    wid = lax.axis_index("s") * NC + lax.axis_index("c")
    base = wid * B_PER_W
    pltpu.sync_copy(users_hbm.at[pl.ds(base, B_PER_W)], uidx_v)
    copies = []
    for c in range(N_CHUNKS):
        s = pl.ds(c * CHUNK, CHUNK)
        copies.append(pltpu.async_copy(tu_hbm.at[uidx_v.at[s]], urows_v.at[s], usem))
    for cp in copies:
        cp.wait()
    pltpu.sync_copy(urows_v,
                    theta_out.at[pl.ds(base, B_PER_W), pl.ds(0, EMBED_K)])


def _sc_gather_tu(users, tu):
    mesh = plsc.VectorSubcoreMesh(core_axis_name="c", subcore_axis_name="s")
    return pl.kernel(
        _sc_gather_tu_body,
        out_type=jax.ShapeDtypeStruct((B, FEAT), jnp.float32),
        mesh=mesh,
        compiler_params=pltpu.CompilerParams(use_tc_tiling_on_sc=False),
        scratch_types=[
            pltpu.VMEM((B_PER_W,), jnp.int32),
            pltpu.VMEM((B_PER_W, EMBED_K), jnp.float32),
            pltpu.SemaphoreType.DMA,
        ],
    )(users, tu)


TC_BLK = 2048


def _tc_body(gut_ref, git_ref, th_ref, ef_ref, w_ref, b_ref,
             xui_ref, projt_ref, thetat_ref):
    e = ef_ref[...]
    mm = lax.dot_general(w_ref[...], e, (((1,), (1,)), ((), ())),
                         preferred_element_type=jnp.float32)
    p = mm + b_ref[...]
    n = jnp.sqrt(jnp.sum(p * p, axis=0, keepdims=True))
    p = p / jnp.maximum(n, 1e-12)
    projt_ref[...] = p
    th = th_ref[...][:, :EMBED_K]
    eye = jnp.eye(EMBED_K, dtype=jnp.float32)
    tht = lax.dot_general(eye, th, (((1,), (1,)), ((), ())),
                          preferred_element_type=jnp.float32)
    thetat_ref[...] = tht
    xui = (jnp.sum(gut_ref[...] * git_ref[...], axis=0, keepdims=True)
           + jnp.sum(tht * p, axis=0, keepdims=True))
    xui_ref[...] = xui


def _tc_compute(gut, git, theta128, effe_i, w, bcol):
    grid = (B // TC_BLK,)
    cm_blk = pl.BlockSpec((EMBED_K, TC_BLK), lambda i: (0, i))
    rm_blk = pl.BlockSpec((TC_BLK, FEAT), lambda i: (i, 0))
    return pl.pallas_call(
        _tc_body,
        grid=grid,
        in_specs=[
            cm_blk,
            cm_blk,
            rm_blk,
            rm_blk,
            pl.BlockSpec((EMBED_K, FEAT), lambda i: (0, 0)),
            pl.BlockSpec((EMBED_K, 1), lambda i: (0, 0)),
        ],
        out_specs=[
            pl.BlockSpec((1, TC_BLK), lambda i: (0, i)),
            cm_blk,
            cm_blk,
        ],
        out_shape=[
            jax.ShapeDtypeStruct((1, B), jnp.float32),
            jax.ShapeDtypeStruct((EMBED_K, B), jnp.float32),
            jax.ShapeDtypeStruct((EMBED_K, B), jnp.float32),
        ],
    )(gut, git, theta128, effe_i, w, bcol)


def kernel(gu, gi, users, items, Tu, F, W, b):
    users32 = users.astype(jnp.int32)
    items32 = items.astype(jnp.int32)
    effe_i, gamut, gamit = _sc_gather_f(items32, F, gu.T, gi.T)
    theta128 = _sc_gather_tu(users32, Tu)
    xui2d, projt, thetat = _tc_compute(
        gamut, gamit, theta128, effe_i, W, b.reshape(EMBED_K, 1))
    xui = xui2d.reshape(B)
    return (xui, gamut.T, gamit.T, thetat.T, projt.T)


# speedup vs baseline: 3.1016x; 3.1016x over previous
import jax
import jax.numpy as jnp
from jax import lax
from jax.experimental import pallas as pl
from jax.experimental.pallas import tpu as pltpu
from jax.experimental.pallas import tpu_sc as plsc

B = 16384
EMBED_K = 64
FEAT = 128

NC = 2
NS = 16
NW = NC * NS
B_PER_W = B // NW
CHUNK = 128
N_CHUNKS = B_PER_W // CHUNK


def _sc_gather_f_body(items_hbm, f_hbm, effe_out, iidx_v, irows_v, isem):
    wid = lax.axis_index("s") * NC + lax.axis_index("c")
    base = wid * B_PER_W
    pltpu.sync_copy(items_hbm.at[pl.ds(base, B_PER_W)], iidx_v)
    copies = []
    for c in range(N_CHUNKS):
        s = pl.ds(c * CHUNK, CHUNK)
        copies.append(pltpu.async_copy(f_hbm.at[iidx_v.at[s]], irows_v.at[s], isem))
    for cp in copies:
        cp.wait()
    pltpu.sync_copy(irows_v, effe_out.at[pl.ds(base, B_PER_W)])


def _sc_gather_f(items, f):
    mesh = plsc.VectorSubcoreMesh(core_axis_name="c", subcore_axis_name="s")
    return pl.kernel(
        _sc_gather_f_body,
        out_type=jax.ShapeDtypeStruct((B, FEAT), jnp.float32),
        mesh=mesh,
        compiler_params=pltpu.CompilerParams(use_tc_tiling_on_sc=False),
        scratch_types=[
            pltpu.VMEM((B_PER_W,), jnp.int32),
            pltpu.VMEM((B_PER_W, FEAT), jnp.float32),
            pltpu.SemaphoreType.DMA,
        ],
    )(items, f)


def _sc_gather_tu_body(users_hbm, tu_hbm, theta_out, uidx_v, urows_v, usem):
    wid = lax.axis_index("s") * NC + lax.axis_index("c")
    base = wid * B_PER_W
    pltpu.sync_copy(users_hbm.at[pl.ds(base, B_PER_W)], uidx_v)
    copies = []
    for c in range(N_CHUNKS):
        s = pl.ds(c * CHUNK, CHUNK)
        copies.append(pltpu.async_copy(tu_hbm.at[uidx_v.at[s]], urows_v.at[s], usem))
    for cp in copies:
        cp.wait()
    pltpu.sync_copy(urows_v,
                    theta_out.at[pl.ds(base, B_PER_W), pl.ds(0, EMBED_K)])


def _sc_gather_tu(users, tu):
    mesh = plsc.VectorSubcoreMesh(core_axis_name="c", subcore_axis_name="s")
    return pl.kernel(
        _sc_gather_tu_body,
        out_type=jax.ShapeDtypeStruct((B, FEAT), jnp.float32),
        mesh=mesh,
        compiler_params=pltpu.CompilerParams(use_tc_tiling_on_sc=False),
        scratch_types=[
            pltpu.VMEM((B_PER_W,), jnp.int32),
            pltpu.VMEM((B_PER_W, EMBED_K), jnp.float32),
            pltpu.SemaphoreType.DMA,
        ],
    )(users, tu)


TC_BLK = 2048


def _tc_body(gut_ref, git_ref, th_ref, ef_ref, w_ref, b_ref,
             xui_ref, projt_ref, thetat_ref, gaut_ref, gait_ref):
    e = ef_ref[...]
    mm = lax.dot_general(w_ref[...], e, (((1,), (1,)), ((), ())),
                         preferred_element_type=jnp.float32)
    p = mm + b_ref[...]
    n = jnp.sqrt(jnp.sum(p * p, axis=0, keepdims=True))
    p = p / jnp.maximum(n, 1e-12)
    projt_ref[...] = p
    th = th_ref[...][:, :EMBED_K]
    eye = jnp.eye(EMBED_K, dtype=jnp.float32)
    tht = lax.dot_general(eye, th, (((1,), (1,)), ((), ())),
                          preferred_element_type=jnp.float32)
    thetat_ref[...] = tht
    gut = gut_ref[...]
    git = git_ref[...]
    gaut_ref[...] = gut
    gait_ref[...] = git
    xui = (jnp.sum(gut * git, axis=0, keepdims=True)
           + jnp.sum(tht * p, axis=0, keepdims=True))
    xui_ref[...] = xui


def _tc_compute(gut, git, theta128, effe_i, w, bcol):
    grid = (B // TC_BLK,)
    cm_blk = pl.BlockSpec((EMBED_K, TC_BLK), lambda i: (0, i))
    rm_blk = pl.BlockSpec((TC_BLK, FEAT), lambda i: (i, 0))
    return pl.pallas_call(
        _tc_body,
        grid=grid,
        in_specs=[
            cm_blk,
            cm_blk,
            rm_blk,
            rm_blk,
            pl.BlockSpec((EMBED_K, FEAT), lambda i: (0, 0)),
            pl.BlockSpec((EMBED_K, 1), lambda i: (0, 0)),
        ],
        out_specs=[
            pl.BlockSpec((1, TC_BLK), lambda i: (0, i)),
            cm_blk,
            cm_blk,
            cm_blk,
            cm_blk,
        ],
        out_shape=[
            jax.ShapeDtypeStruct((1, B), jnp.float32),
            jax.ShapeDtypeStruct((EMBED_K, B), jnp.float32),
            jax.ShapeDtypeStruct((EMBED_K, B), jnp.float32),
            jax.ShapeDtypeStruct((EMBED_K, B), jnp.float32),
            jax.ShapeDtypeStruct((EMBED_K, B), jnp.float32),
        ],
    )(gut, git, theta128, effe_i, w, bcol)


def kernel(gu, gi, users, items, Tu, F, W, b):
    users32 = users.astype(jnp.int32)
    items32 = items.astype(jnp.int32)
    effe_i = _sc_gather_f(items32, F)
    theta128 = _sc_gather_tu(users32, Tu)
    xui2d, projt, thetat, gaut, gait = _tc_compute(
        gu.T, gi.T, theta128, effe_i, W, b.reshape(EMBED_K, 1))
    xui = xui2d.reshape(B)
    return (xui, gaut.T, gait.T, thetat.T, projt.T)
